# baseline (speedup 1.0000x reference)
import jax
import jax.numpy as jnp
from jax import lax
from jax.experimental import pallas as pl
from jax.experimental.pallas import tpu as pltpu
from jax.experimental.pallas import tpu_sc as plsc

_NC = 2
_NS = 16
_NW = _NC * _NS
_L = 16
_C = 96


def _tec_body(pairs_hbm, atom_hbm, bond_hbm, out_hbm,
              idx0a, idx0b, idx1a, idx1b, rows0a, rows0b, rows1a, rows1b,
              comb0, comb1, sg0, sg1, sb0, sb1, ss0, ss1, si0, si1):
    wid = lax.axis_index("s") * _NC + lax.axis_index("c")
    n_bond = out_hbm.shape[0]
    n_chunks = n_bond // _C
    n_rounds = (n_chunks + _NW - 1) // _NW
    n_rounds = n_rounds + (n_rounds % 2)
    d = atom_hbm.shape[1]
    bufs = (((idx0a, idx0b), (rows0a, rows0b), comb0, sg0, sb0, ss0, si0),
            ((idx1a, idx1b), (rows1a, rows1b), comb1, sg1, sb1, ss1, si1))

    def start_idx(k, b):
        (idx_a, idx_b), _, _, _, _, _, si = bufs[b]
        ci = wid + k * _NW

        @pl.when(ci < n_chunks)
        def _():
            base = ci * _C
            pltpu.async_copy(pairs_hbm.at[pl.ds(base, _C)], idx_a, si)
            pltpu.async_copy(pairs_hbm.at[pl.ds(n_bond + base, _C)], idx_b, si)

    def start_gather(k, b):
        (idx_a, idx_b), (rows_a, rows_b), _, sg, _, _, si = bufs[b]
        ci = wid + k * _NW

        @pl.when(ci < n_chunks)
        def _():
            base = ci * _C
            pltpu.make_async_copy(pairs_hbm.at[pl.ds(base, _C)], idx_a, si).wait()
            pltpu.make_async_copy(pairs_hbm.at[pl.ds(n_bond + base, _C)],
                                  idx_b, si).wait()
            pltpu.async_copy(atom_hbm.at[idx_a], rows_a, sg)
            pltpu.async_copy(atom_hbm.at[idx_b], rows_b, sg)

    def start_bond(k, b):
        pass

    def finish(k, b):
        (idx_a, idx_b), (rows_a, rows_b), comb_v, sg, sb, ss, _ = bufs[b]
        ci = wid + k * _NW

        @pl.when(ci < n_chunks)
        def _():
            base = ci * _C
            pltpu.make_async_copy(atom_hbm.at[idx_a], rows_a, sg).wait()
            pltpu.make_async_copy(atom_hbm.at[idx_b], rows_b, sg).wait()

            @plsc.parallel_loop(0, _C, unroll=4)
            def bond_body(i):
                for j in range(d // _L):
                    sl = pl.ds(j * _L, _L)
                    a0 = rows_a[i, sl]
                    a1 = rows_b[i, sl]
                    comb_v[i, pl.ds(d + j * _L, _L)] = (a0 + a1) * 0.5
                    comb_v[i, pl.ds(2 * d + j * _L, _L)] = jnp.abs(a0 - a1)

            pltpu.async_copy(comb_v.at[:, pl.ds(d, 2 * d)],
                             out_hbm.at[pl.ds(base, _C), pl.ds(d, 2 * d)], ss)

    def wait_store(k, b):
        _, _, comb_v, _, _, ss, _ = bufs[b]
        ci = wid + k * _NW

        @pl.when(ci < n_chunks)
        def _():
            base = ci * _C
            pltpu.make_async_copy(comb_v.at[:, pl.ds(d, 2 * d)],
                                  out_hbm.at[pl.ds(base, _C), pl.ds(d, 2 * d)],
                                  ss).wait()

    start_idx(0, 0)
    start_idx(1, 1)
    start_gather(0, 0)
    start_gather(1, 1)
    start_bond(0, 0)
    start_bond(1, 1)

    def round_body(k2, carry):
        k = 2 * k2
        finish(k, 0)
        start_idx(k + 2, 0)
        finish(k + 1, 1)
        start_idx(k + 3, 1)
        start_gather(k + 2, 0)
        wait_store(k, 0)
        start_bond(k + 2, 0)
        start_gather(k + 3, 1)
        wait_store(k + 1, 1)
        start_bond(k + 3, 1)
        return carry

    lax.fori_loop(0, n_rounds // 2, round_body, 0)


def kernel(x_atom, x_bond, atom_pairs):
    n_bond, d = x_bond.shape
    pairs_flat = atom_pairs.T.reshape(-1)
    mesh = plsc.VectorSubcoreMesh(core_axis_name="c", subcore_axis_name="s")
    out = pl.kernel(
        _tec_body,
        out_type=jax.ShapeDtypeStruct((n_bond, 3 * d), jnp.float32),
        mesh=mesh,
        scratch_types=[
            pltpu.VMEM((_C,), jnp.int32),
            pltpu.VMEM((_C,), jnp.int32),
            pltpu.VMEM((_C,), jnp.int32),
            pltpu.VMEM((_C,), jnp.int32),
            pltpu.VMEM((_C, d), jnp.float32),
            pltpu.VMEM((_C, d), jnp.float32),
            pltpu.VMEM((_C, d), jnp.float32),
            pltpu.VMEM((_C, d), jnp.float32),
            pltpu.VMEM((_C, 3 * d), jnp.float32),
            pltpu.VMEM((_C, 3 * d), jnp.float32),
            pltpu.SemaphoreType.DMA,
            pltpu.SemaphoreType.DMA,
            pltpu.SemaphoreType.DMA,
            pltpu.SemaphoreType.DMA,
            pltpu.SemaphoreType.DMA,
            pltpu.SemaphoreType.DMA,
            pltpu.SemaphoreType.DMA,
            pltpu.SemaphoreType.DMA,
        ],
    )(pairs_flat, x_atom, x_bond)
    return out.at[:, 0:d].set(x_bond)


# speedup vs baseline: 1.1040x; 1.1040x over previous
import jax
import jax.numpy as jnp
from jax import lax
from jax.experimental import pallas as pl
from jax.experimental.pallas import tpu as pltpu
from jax.experimental.pallas import tpu_sc as plsc

_NC = 2
_NS = 16
_NW = _NC * _NS
_L = 16
_C = 96


def _tec_body(pairs_hbm, atom_hbm, bond_hbm, out_hbm,
              idx0a, idx0b, idx1a, idx1b, rows0a, rows0b, rows1a, rows1b,
              comb0, comb1, sg0, sg1, sb0, sb1, ss0, ss1, si0, si1):
    wid = lax.axis_index("s") * _NC + lax.axis_index("c")
    n_bond = out_hbm.shape[0]
    n_chunks = n_bond // _C
    n_rounds = (n_chunks + _NW - 1) // _NW
    n_rounds = n_rounds + (n_rounds % 2)
    d = atom_hbm.shape[1]
    bufs = (((idx0a, idx0b), (rows0a, rows0b), comb0, sg0, sb0, ss0, si0),
            ((idx1a, idx1b), (rows1a, rows1b), comb1, sg1, sb1, ss1, si1))

    def start_idx(k, b):
        (idx_a, idx_b), _, _, _, _, _, si = bufs[b]
        ci = wid + k * _NW

        @pl.when(ci < n_chunks)
        def _():
            base = ci * _C
            pltpu.async_copy(pairs_hbm.at[pl.ds(base, _C)], idx_a, si)
            pltpu.async_copy(pairs_hbm.at[pl.ds(n_bond + base, _C)], idx_b, si)

    def start_gather(k, b):
        (idx_a, idx_b), (rows_a, rows_b), _, sg, _, _, si = bufs[b]
        ci = wid + k * _NW

        @pl.when(ci < n_chunks)
        def _():
            base = ci * _C
            pltpu.make_async_copy(pairs_hbm.at[pl.ds(base, _C)], idx_a, si).wait()
            pltpu.make_async_copy(pairs_hbm.at[pl.ds(n_bond + base, _C)],
                                  idx_b, si).wait()
            pltpu.async_copy(atom_hbm.at[idx_a], rows_a, sg)
            pltpu.async_copy(atom_hbm.at[idx_b], rows_b, sg)

    def start_bond(k, b):
        pass

    def finish(k, b):
        (idx_a, idx_b), (rows_a, rows_b), comb_v, sg, sb, ss, _ = bufs[b]
        ci = wid + k * _NW

        @pl.when(ci < n_chunks)
        def _():
            base = ci * _C
            pltpu.make_async_copy(atom_hbm.at[idx_a], rows_a, sg).wait()
            pltpu.make_async_copy(atom_hbm.at[idx_b], rows_b, sg).wait()

            @plsc.parallel_loop(0, _C, unroll=4)
            def bond_body(i):
                for j in range(d // _L):
                    sl = pl.ds(j * _L, _L)
                    a0 = rows_a[i, sl]
                    a1 = rows_b[i, sl]
                    comb_v[i, pl.ds(d + j * _L, _L)] = (a0 + a1) * 0.5
                    comb_v[i, pl.ds(2 * d + j * _L, _L)] = jnp.abs(a0 - a1)

            pltpu.async_copy(comb_v.at[:, pl.ds(d, 2 * d)],
                             out_hbm.at[pl.ds(base, _C), pl.ds(d, 2 * d)], ss)

    def wait_store(k, b):
        _, _, comb_v, _, _, ss, _ = bufs[b]
        ci = wid + k * _NW

        @pl.when(ci < n_chunks)
        def _():
            base = ci * _C
            pltpu.make_async_copy(comb_v.at[:, pl.ds(d, 2 * d)],
                                  out_hbm.at[pl.ds(base, _C), pl.ds(d, 2 * d)],
                                  ss).wait()

    start_idx(0, 0)
    start_idx(1, 1)
    start_gather(0, 0)
    start_gather(1, 1)
    start_bond(0, 0)
    start_bond(1, 1)

    def round_body(k2, carry):
        k = 2 * k2
        finish(k, 0)
        start_idx(k + 2, 0)
        finish(k + 1, 1)
        start_idx(k + 3, 1)
        start_gather(k + 2, 0)
        wait_store(k, 0)
        start_bond(k + 2, 0)
        start_gather(k + 3, 1)
        wait_store(k + 1, 1)
        start_bond(k + 3, 1)
        return carry

    lax.fori_loop(0, n_rounds // 2, round_body, 0)


def kernel(x_atom, x_bond, atom_pairs):
    n_bond, d = x_bond.shape
    pairs_flat = atom_pairs.T.reshape(-1)
    mesh = plsc.VectorSubcoreMesh(core_axis_name="c", subcore_axis_name="s")
    out = pl.kernel(
        _tec_body,
        out_type=jax.ShapeDtypeStruct((n_bond, 3 * d), jnp.float32),
        mesh=mesh,
        scratch_types=[
            pltpu.VMEM((_C,), jnp.int32),
            pltpu.VMEM((_C,), jnp.int32),
            pltpu.VMEM((_C,), jnp.int32),
            pltpu.VMEM((_C,), jnp.int32),
            pltpu.VMEM((_C, d), jnp.float32),
            pltpu.VMEM((_C, d), jnp.float32),
            pltpu.VMEM((_C, d), jnp.float32),
            pltpu.VMEM((_C, d), jnp.float32),
            pltpu.VMEM((_C, 3 * d), jnp.float32),
            pltpu.VMEM((_C, 3 * d), jnp.float32),
            pltpu.SemaphoreType.DMA,
            pltpu.SemaphoreType.DMA,
            pltpu.SemaphoreType.DMA,
            pltpu.SemaphoreType.DMA,
            pltpu.SemaphoreType.DMA,
            pltpu.SemaphoreType.DMA,
            pltpu.SemaphoreType.DMA,
            pltpu.SemaphoreType.DMA,
        ],
    )(pairs_flat, x_atom, x_bond)

    blk = 4000
    def _fill(xb_ref, _outin_ref, o_ref):
        o_ref[...] = xb_ref[...]

    out = pl.pallas_call(
        _fill,
        grid=(n_bond // blk,),
        in_specs=[
            pl.BlockSpec((blk, d), lambda i: (i, 0)),
            pl.BlockSpec(memory_space=pl.ANY),
        ],
        out_specs=pl.BlockSpec((blk, d), lambda i: (i, 0)),
        out_shape=jax.ShapeDtypeStruct((n_bond, 3 * d), jnp.float32),
        input_output_aliases={1: 0},
    )(x_bond, out)
    return out


# speedup vs baseline: 1.1596x; 1.0503x over previous
import jax
import jax.numpy as jnp
from jax import lax
from jax.experimental import pallas as pl
from jax.experimental.pallas import tpu as pltpu
from jax.experimental.pallas import tpu_sc as plsc

_NC = 2
_NS = 16
_NW = _NC * _NS
_L = 16
_C = 80


def _tec_body(pairs_hbm, atom_hbm, bond_hbm, out_hbm,
              idx0a, idx0b, idx1a, idx1b, rows0a, rows0b, rows1a, rows1b,
              comb0, comb1, sg0, sg1, sb0, sb1, ss0, ss1, si0, si1):
    wid = lax.axis_index("s") * _NC + lax.axis_index("c")
    n_bond = out_hbm.shape[0]
    n_chunks = n_bond // _C
    n_rounds = (n_chunks + _NW - 1) // _NW
    n_rounds = n_rounds + (n_rounds % 2)
    d = atom_hbm.shape[1]
    bufs = (((idx0a, idx0b), (rows0a, rows0b), comb0, sg0, sb0, ss0, si0),
            ((idx1a, idx1b), (rows1a, rows1b), comb1, sg1, sb1, ss1, si1))

    def start_idx(k, b):
        (idx_a, idx_b), _, _, _, _, _, si = bufs[b]
        ci = wid + k * _NW

        @pl.when(ci < n_chunks)
        def _():
            base = ci * _C
            pltpu.async_copy(pairs_hbm.at[pl.ds(base, _C)], idx_a, si)
            pltpu.async_copy(pairs_hbm.at[pl.ds(n_bond + base, _C)], idx_b, si)

    def start_gather(k, b):
        (idx_a, idx_b), (rows_a, rows_b), _, sg, _, _, si = bufs[b]
        ci = wid + k * _NW

        @pl.when(ci < n_chunks)
        def _():
            base = ci * _C
            pltpu.make_async_copy(pairs_hbm.at[pl.ds(base, _C)], idx_a, si).wait()
            pltpu.make_async_copy(pairs_hbm.at[pl.ds(n_bond + base, _C)],
                                  idx_b, si).wait()
            pltpu.async_copy(atom_hbm.at[idx_a], rows_a, sg)
            pltpu.async_copy(atom_hbm.at[idx_b], rows_b, sg)

    def start_bond(k, b):
        _, _, comb_v, _, sb, _, _ = bufs[b]
        ci = wid + k * _NW

        @pl.when(ci < n_chunks)
        def _():
            base = ci * _C
            pltpu.async_copy(bond_hbm.at[pl.ds(base, _C)],
                             comb_v.at[:, pl.ds(0, d)], sb)

    def finish(k, b):
        (idx_a, idx_b), (rows_a, rows_b), comb_v, sg, sb, ss, _ = bufs[b]
        ci = wid + k * _NW

        @pl.when(ci < n_chunks)
        def _():
            base = ci * _C
            pltpu.make_async_copy(atom_hbm.at[idx_a], rows_a, sg).wait()
            pltpu.make_async_copy(atom_hbm.at[idx_b], rows_b, sg).wait()

            @plsc.parallel_loop(0, _C, unroll=4)
            def bond_body(i):
                for j in range(d // _L):
                    sl = pl.ds(j * _L, _L)
                    a0 = rows_a[i, sl]
                    a1 = rows_b[i, sl]
                    comb_v[i, pl.ds(d + j * _L, _L)] = (a0 + a1) * 0.5
                    comb_v[i, pl.ds(2 * d + j * _L, _L)] = jnp.abs(a0 - a1)

            pltpu.make_async_copy(bond_hbm.at[pl.ds(base, _C)],
                                  comb_v.at[:, pl.ds(0, d)], sb).wait()
            pltpu.async_copy(comb_v, out_hbm.at[pl.ds(base, _C)], ss)

    def wait_store(k, b):
        _, _, comb_v, _, _, ss, _ = bufs[b]
        ci = wid + k * _NW

        @pl.when(ci < n_chunks)
        def _():
            base = ci * _C
            pltpu.make_async_copy(comb_v, out_hbm.at[pl.ds(base, _C)], ss).wait()

    start_idx(0, 0)
    start_idx(1, 1)
    start_gather(0, 0)
    start_gather(1, 1)
    start_bond(0, 0)
    start_bond(1, 1)

    def round_body(k2, carry):
        k = 2 * k2
        finish(k, 0)
        start_idx(k + 2, 0)
        finish(k + 1, 1)
        start_idx(k + 3, 1)
        start_gather(k + 2, 0)
        wait_store(k, 0)
        start_bond(k + 2, 0)
        start_gather(k + 3, 1)
        wait_store(k + 1, 1)
        start_bond(k + 3, 1)
        return carry

    lax.fori_loop(0, n_rounds // 2, round_body, 0)


def kernel(x_atom, x_bond, atom_pairs):
    n_bond, d = x_bond.shape
    pairs_flat = atom_pairs.T.reshape(-1)
    mesh = plsc.VectorSubcoreMesh(core_axis_name="c", subcore_axis_name="s")
    out = pl.kernel(
        _tec_body,
        out_type=jax.ShapeDtypeStruct((n_bond, 3 * d), jnp.float32),
        mesh=mesh,
        scratch_types=[
            pltpu.VMEM((_C,), jnp.int32),
            pltpu.VMEM((_C,), jnp.int32),
            pltpu.VMEM((_C,), jnp.int32),
            pltpu.VMEM((_C,), jnp.int32),
            pltpu.VMEM((_C, d), jnp.float32),
            pltpu.VMEM((_C, d), jnp.float32),
            pltpu.VMEM((_C, d), jnp.float32),
            pltpu.VMEM((_C, d), jnp.float32),
            pltpu.VMEM((_C, 3 * d), jnp.float32),
            pltpu.VMEM((_C, 3 * d), jnp.float32),
            pltpu.SemaphoreType.DMA,
            pltpu.SemaphoreType.DMA,
            pltpu.SemaphoreType.DMA,
            pltpu.SemaphoreType.DMA,
            pltpu.SemaphoreType.DMA,
            pltpu.SemaphoreType.DMA,
            pltpu.SemaphoreType.DMA,
            pltpu.SemaphoreType.DMA,
        ],
    )(pairs_flat, x_atom, x_bond)
    return out
